# Initial kernel scaffold; baseline (speedup 1.0000x reference)
import jax
import jax.numpy as jnp
from jax.experimental import pallas as pl


def kernel(x, edge_index, edge, W0, b0, W1, att_src1, att_dst1, bias1, W2, att_src2, att_dst2, bias2):
    raise NotImplementedError("write your pallas kernel here")


# speedup vs baseline: 9.0564x; 9.0564x over previous
import functools

import jax
import jax.numpy as jnp
from jax import lax
from jax.experimental import pallas as pl
from jax.experimental.pallas import tpu as pltpu
from jax.experimental.pallas import tpu_sc as plsc

_N = 10000
_NPAD = 10240
_D = 128
_H1 = 256
_H2 = 128

_E = 320000
_EAUG = _E + _N
_KB = 128
_TILE_E = 162 * _KB
_EPAD = 16 * _TILE_E
_TILE_E2 = 79 * _KB
_EPAD2 = 32 * _TILE_E2
_SD_PAD = 2228224
_ROWS_T = _NPAD // 16

_R = 1024


def _dense1_body(x_ref, w0_ref, b0_ref, w1_ref, s_ref, d_ref, g_ref, a_ref):
    h = jnp.dot(x_ref[...], w0_ref[...], preferred_element_type=jnp.float32)
    h = jnp.maximum(h + b0_ref[0], 0.0)
    g = jnp.dot(h, w1_ref[...], preferred_element_type=jnp.float32)
    for q in range(4):
        g_ref[q] = g[:, q * 64:(q + 1) * 64]
    a_ref[0] = jnp.sum(g * s_ref[0], axis=1)
    a_ref[1] = jnp.sum(g * d_ref[0], axis=1)


def _dense1(x_pad, W0, b0, W1, att_src1, att_dst1):
    return pl.pallas_call(
        _dense1_body,
        grid=(_NPAD // _R,),
        in_specs=[
            pl.BlockSpec((_R, _D), lambda i: (i, 0)),
            pl.BlockSpec((_D, 256), lambda i: (0, 0)),
            pl.BlockSpec((1, 256), lambda i: (0, 0)),
            pl.BlockSpec((256, _H1), lambda i: (0, 0)),
            pl.BlockSpec((1, _H1), lambda i: (0, 0)),
            pl.BlockSpec((1, _H1), lambda i: (0, 0)),
        ],
        out_specs=[
            pl.BlockSpec((4, _R, 64), lambda i: (0, i, 0)),
            pl.BlockSpec((2, _R), lambda i: (0, i)),
        ],
        out_shape=[
            jax.ShapeDtypeStruct((4, _NPAD, 64), jnp.float32),
            jax.ShapeDtypeStruct((2, _NPAD), jnp.float32),
        ],
    )(x_pad, W0, b0[None], W1, att_src1[None], att_dst1[None])


def _dense2_body(na_ref, nb_ref, den_ref, b1_ref, w2_ref, s_ref, d_ref,
                 g_ref, a_ref):
    inv = 1.0 / (den_ref[:, :1] + 1e-16)
    g = None
    for q in range(4):
        nq = na_ref[q] if q < 2 else nb_ref[q - 2]
        hq = jnp.maximum(nq * inv + b1_ref[0, q * 64:(q + 1) * 64], 0.0)
        gq = jnp.dot(hq, w2_ref[q * 64:(q + 1) * 64, :],
                     preferred_element_type=jnp.float32)
        g = gq if g is None else g + gq
    g_ref[0] = g[:, :64]
    g_ref[1] = g[:, 64:]
    a_ref[0] = jnp.sum(g * s_ref[0], axis=1)
    a_ref[1] = jnp.sum(g * d_ref[0], axis=1)


def _dense2(numa, numb, den1, bias1, W2, att_src2, att_dst2):
    return pl.pallas_call(
        _dense2_body,
        grid=(_NPAD // _R,),
        in_specs=[
            pl.BlockSpec((2, _R, 64), lambda i: (0, i, 0)),
            pl.BlockSpec((2, _R, 64), lambda i: (0, i, 0)),
            pl.BlockSpec((_R, 16), lambda i: (i, 0)),
            pl.BlockSpec((1, _H1), lambda i: (0, 0)),
            pl.BlockSpec((_H1, _H2), lambda i: (0, 0)),
            pl.BlockSpec((1, _H2), lambda i: (0, 0)),
            pl.BlockSpec((1, _H2), lambda i: (0, 0)),
        ],
        out_specs=[
            pl.BlockSpec((2, _R, 64), lambda i: (0, i, 0)),
            pl.BlockSpec((2, _R), lambda i: (0, i)),
        ],
        out_shape=[
            jax.ShapeDtypeStruct((2, _NPAD, 64), jnp.float32),
            jax.ShapeDtypeStruct((2, _NPAD), jnp.float32),
        ],
    )(numa, numb, den1, bias1[None], W2, att_src2[None], att_dst2[None])


def _dense3_body(num_ref, den_ref, b2_ref, h_ref):
    inv = 1.0 / (den_ref[:, :1] + 1e-16)
    h_ref[...] = jnp.concatenate(
        [num_ref[0] * inv + b2_ref[0, :64],
         num_ref[1] * inv + b2_ref[0, 64:]], axis=1)


def _dense3(num2, den2, bias2):
    return pl.pallas_call(
        _dense3_body,
        grid=(_NPAD // _R,),
        in_specs=[
            pl.BlockSpec((2, _R, 64), lambda i: (0, i, 0)),
            pl.BlockSpec((_R, 16), lambda i: (i, 0)),
            pl.BlockSpec((1, _H2), lambda i: (0, 0)),
        ],
        out_specs=pl.BlockSpec((_R, _H2), lambda i: (i, 0)),
        out_shape=jax.ShapeDtypeStruct((_NPAD, _H2), jnp.float32),
    )(num2, den2, bias2[None])


def _make_gat_edge(width):
    uw = width // 16
    nblk = _TILE_E // _KB
    arows = _NPAD // width
    abits = width.bit_length() - 1
    mesh = plsc.VectorSubcoreMesh(core_axis_name="c", subcore_axis_name="s")

    @functools.partial(
        pl.kernel, mesh=mesh,
        compiler_params=pltpu.CompilerParams(needs_layout_passes=False,
                                             use_tc_tiling_on_sc=False),
        out_type=[jax.ShapeDtypeStruct((2 * _NPAD, width), jnp.float32),
                  jax.ShapeDtypeStruct((_NPAD, 16), jnp.float32)],
        scratch_types=[
            pltpu.VMEM((arows, width), jnp.float32),
            pltpu.VMEM((arows, width), jnp.float32),
            pltpu.VMEM((_KB,), jnp.int32),
            pltpu.VMEM((_KB,), jnp.int32),
            pltpu.VMEM((_KB,), jnp.int32),
            pltpu.VMEM((_KB,), jnp.float32),
            pltpu.VMEM((_KB, 16), jnp.float32),
            pltpu.VMEM((_KB, width), jnp.float32),
            pltpu.VMEM_SHARED((_NPAD, width), jnp.float32),
            pltpu.VMEM_SHARED((_NPAD, 16), jnp.float32),
            pltpu.SemaphoreType.DMA,
        ],
    )
    def k(g_hbm, sd_hbm, out_hbm, den_hbm,
          as_v, ad_v, pi_v, si_v, di_v, w_v, wc_v, rows_v, acc_sh, den_sh, sem):
        c = lax.axis_index("c")
        s = lax.axis_index("s")

        def zero_body(r, _):
            for u in range(uw):
                rows_v[r, pl.ds(u * 16, 16)] = jnp.zeros((16,), jnp.float32)
            wc_v[r, pl.ds(0, 16)] = jnp.zeros((16,), jnp.float32)
            return 0
        lax.fori_loop(0, _KB, zero_body, 0)
        for i in range(_ROWS_T // _KB):
            pltpu.sync_copy(rows_v, acc_sh.at[pl.ds(s * _ROWS_T + i * _KB, _KB)])
            pltpu.sync_copy(wc_v, den_sh.at[pl.ds(s * _ROWS_T + i * _KB, _KB)])
        pltpu.sync_copy(g_hbm.at[pl.ds(2 * _NPAD, arows)], as_v)
        pltpu.sync_copy(g_hbm.at[pl.ds(2 * _NPAD + arows, arows)], ad_v)
        plsc.subcore_barrier()

        def block(b, _):
            base = s * _TILE_E + b * _KB
            pltpu.sync_copy(sd_hbm.at[pl.ds(base, _KB)], pi_v)
            for j in range(_KB // 16):
                p = pi_v[pl.ds(j * 16, 16)]
                sidx = lax.shift_right_logical(p, 14)
                didx = lax.bitwise_and(p, 16383)
                di_v[pl.ds(j * 16, 16)] = didx
                lo = lax.bitwise_and(sidx, width - 1)
                t = plsc.load_gather(as_v, [lax.shift_right_logical(sidx, abits), lo])
                lo = lax.bitwise_and(didx, width - 1)
                t = t + plsc.load_gather(ad_v, [lax.shift_right_logical(didx, abits), lo])
                t = jnp.where(t >= 0.0, t, t * 0.2)
                w = jnp.exp(t)
                eid = base + j * 16 + lax.broadcasted_iota(jnp.int32, (16,), 0)
                w = jnp.where(eid < _EAUG, w, 0.0)
                w_v[pl.ds(j * 16, 16)] = w
                si_v[pl.ds(j * 16, 16)] = sidx + c * _NPAD
            copy = pltpu.async_copy(g_hbm.at[si_v], rows_v, sem)
            copy.wait()

            def scale(r, _):
                wspl = plsc.load_gather(w_v, [jnp.full((16,), r, jnp.int32)])
                wc_v[r, pl.ds(0, 16)] = wspl
                for u in range(uw):
                    rows_v[r, pl.ds(u * 16, 16)] = rows_v[r, pl.ds(u * 16, 16)] * wspl
                return 0
            lax.fori_loop(0, _KB, scale, 0)
            pltpu.sync_copy(rows_v, acc_sh.at[di_v], add=True)
            pltpu.sync_copy(wc_v, den_sh.at[di_v], add=True)
            return 0
        lax.fori_loop(0, nblk, block, 0)
        plsc.subcore_barrier()

        pltpu.sync_copy(acc_sh.at[pl.ds(s * _ROWS_T, _ROWS_T)],
                        out_hbm.at[pl.ds(c * _NPAD + s * _ROWS_T, _ROWS_T)])

        @pl.when(c == 0)
        def _():
            pltpu.sync_copy(den_sh.at[pl.ds(s * _ROWS_T, _ROWS_T)],
                            den_hbm.at[pl.ds(s * _ROWS_T, _ROWS_T)])

    return k


_gat_edge_64 = _make_gat_edge(64)


def _make_edge_dot():
    nblk = _TILE_E2 // _KB
    mesh = plsc.VectorSubcoreMesh(core_axis_name="c", subcore_axis_name="s")

    @functools.partial(
        pl.kernel, mesh=mesh,
        compiler_params=pltpu.CompilerParams(needs_layout_passes=False),
        out_type=jax.ShapeDtypeStruct((_EPAD2,), jnp.float32),
        scratch_types=[
            pltpu.VMEM((_KB,), jnp.int32),
            pltpu.VMEM((_KB,), jnp.int32),
            pltpu.VMEM((_KB,), jnp.int32),
            pltpu.VMEM((_KB, _H2), jnp.float32),
            pltpu.VMEM((_KB, _H2), jnp.float32),
            pltpu.VMEM((_KB,), jnp.float32),
            pltpu.SemaphoreType.DMA,
        ],
    )
    def k(h_hbm, ee_hbm, out_hbm, pi_v, i0_v, i1_v, r0_v, r1_v, dots_v, sem):
        c = lax.axis_index("c")
        s = lax.axis_index("s")
        wid = s * 2 + c
        lane0 = lax.broadcasted_iota(jnp.int32, (16,), 0) == 0

        def block(b, _):
            base = wid * _TILE_E2 + b * _KB
            pltpu.sync_copy(ee_hbm.at[pl.ds(base, _KB)], pi_v)
            for j in range(_KB // 16):
                p = pi_v[pl.ds(j * 16, 16)]
                i0_v[pl.ds(j * 16, 16)] = lax.shift_right_logical(p, 14)
                i1_v[pl.ds(j * 16, 16)] = lax.bitwise_and(p, 16383)
            pltpu.async_copy(h_hbm.at[i0_v], r0_v, sem).wait()
            pltpu.async_copy(h_hbm.at[i1_v], r1_v, sem).wait()

            def dot(r, _):
                acc = r0_v[r, pl.ds(0, 16)] * r1_v[r, pl.ds(0, 16)]
                for u in range(1, _H2 // 16):
                    acc = acc + r0_v[r, pl.ds(u * 16, 16)] * r1_v[r, pl.ds(u * 16, 16)]
                t = jnp.sum(acc)
                plsc.store_scatter(dots_v, [jnp.full((16,), r, jnp.int32)],
                                   jnp.full((16,), t, jnp.float32), mask=lane0)
                return 0
            lax.fori_loop(0, _KB, dot, 0)
            for j in range(_KB // 16):
                v = dots_v[pl.ds(j * 16, 16)]
                dots_v[pl.ds(j * 16, 16)] = 1.0 / (1.0 + jnp.exp(-v))
            pltpu.sync_copy(dots_v, out_hbm.at[pl.ds(base, _KB)])
            return 0
        lax.fori_loop(0, nblk, block, 0)

    return k


_edge_dot = _make_edge_dot()


def kernel(x, edge_index, edge, W0, b0, W1, att_src1, att_dst1, bias1,
           W2, att_src2, att_dst2, bias2):
    loop = jnp.arange(_N, dtype=jnp.int32)
    src = jnp.concatenate([edge_index[0].astype(jnp.int32), loop])
    dst = jnp.concatenate([edge_index[1].astype(jnp.int32), loop])
    sd_p = jnp.pad(src * 16384 + dst, (0, _SD_PAD - _EAUG))
    x_p = jnp.pad(x, ((0, _NPAD - _N), (0, 0)))

    g1, a1 = _dense1(x_p, W0, b0, W1, att_src1, att_dst1)
    a1_rows = a1.reshape(2 * (_NPAD // 64), 64)
    table_a = jnp.concatenate([g1[0], g1[1], a1_rows], axis=0)
    table_b = jnp.concatenate([g1[2], g1[3], a1_rows], axis=0)
    num_a, den1 = _gat_edge_64(table_a, sd_p)
    num_b, _den_dup = _gat_edge_64(table_b, sd_p)
    g2, a2 = _dense2(num_a.reshape(2, _NPAD, 64), num_b.reshape(2, _NPAD, 64),
                     den1, bias1, W2, att_src2, att_dst2)
    table_2 = jnp.concatenate([g2[0], g2[1],
                               a2.reshape(2 * (_NPAD // 64), 64)], axis=0)
    num2, den2 = _gat_edge_64(table_2, sd_p)
    h2 = _dense3(num2.reshape(2, _NPAD, 64), den2, bias2)

    ee = edge[0].astype(jnp.int32) * 16384 + edge[1].astype(jnp.int32)
    feats = _edge_dot(h2, jnp.pad(ee, (0, _EPAD2 - _E)))
    return feats[:_E]


# speedup vs baseline: 9.4203x; 1.0402x over previous
import functools

import jax
import jax.numpy as jnp
from jax import lax
from jax.experimental import pallas as pl
from jax.experimental.pallas import tpu as pltpu
from jax.experimental.pallas import tpu_sc as plsc

_N = 10000
_NPAD = 10240
_D = 128
_H1 = 256
_H2 = 128

_E = 320000
_EAUG = _E + _N
_KB = 128
_G = 4
_BLK = _G * _KB
_TILE_E = 41 * _BLK
_EPAD = 16 * _TILE_E
_TILE_E2 = 79 * _KB
_EPAD2 = 32 * _TILE_E2
_SD_PAD = 2228224
_ROWS_T = _NPAD // 16

_R = 1024


def _dense1_body(x_ref, w0_ref, b0_ref, w1_ref, s_ref, d_ref, g_ref, a_ref):
    h = jnp.dot(x_ref[...], w0_ref[...], preferred_element_type=jnp.float32)
    h = jnp.maximum(h + b0_ref[0], 0.0)
    g = jnp.dot(h, w1_ref[...], preferred_element_type=jnp.float32)
    for q in range(4):
        g_ref[q] = g[:, q * 64:(q + 1) * 64]
    a_ref[0] = jnp.sum(g * s_ref[0], axis=1)
    a_ref[1] = jnp.sum(g * d_ref[0], axis=1)


def _dense1(x_pad, W0, b0, W1, att_src1, att_dst1):
    return pl.pallas_call(
        _dense1_body,
        grid=(_NPAD // _R,),
        in_specs=[
            pl.BlockSpec((_R, _D), lambda i: (i, 0)),
            pl.BlockSpec((_D, 256), lambda i: (0, 0)),
            pl.BlockSpec((1, 256), lambda i: (0, 0)),
            pl.BlockSpec((256, _H1), lambda i: (0, 0)),
            pl.BlockSpec((1, _H1), lambda i: (0, 0)),
            pl.BlockSpec((1, _H1), lambda i: (0, 0)),
        ],
        out_specs=[
            pl.BlockSpec((4, _R, 64), lambda i: (0, i, 0)),
            pl.BlockSpec((2, _R), lambda i: (0, i)),
        ],
        out_shape=[
            jax.ShapeDtypeStruct((4, _NPAD, 64), jnp.float32),
            jax.ShapeDtypeStruct((2, _NPAD), jnp.float32),
        ],
    )(x_pad, W0, b0[None], W1, att_src1[None], att_dst1[None])


def _dense2_body(na_ref, nb_ref, den_ref, b1_ref, w2_ref, s_ref, d_ref,
                 g_ref, a_ref):
    inv = 1.0 / (den_ref[:, :1] + 1e-16)
    g = None
    for q in range(4):
        nq = na_ref[q] if q < 2 else nb_ref[q - 2]
        hq = jnp.maximum(nq * inv + b1_ref[0, q * 64:(q + 1) * 64], 0.0)
        gq = jnp.dot(hq, w2_ref[q * 64:(q + 1) * 64, :],
                     preferred_element_type=jnp.float32)
        g = gq if g is None else g + gq
    g_ref[0] = g[:, :64]
    g_ref[1] = g[:, 64:]
    a_ref[0] = jnp.sum(g * s_ref[0], axis=1)
    a_ref[1] = jnp.sum(g * d_ref[0], axis=1)


def _dense2(numa, numb, den1, bias1, W2, att_src2, att_dst2):
    return pl.pallas_call(
        _dense2_body,
        grid=(_NPAD // _R,),
        in_specs=[
            pl.BlockSpec((2, _R, 64), lambda i: (0, i, 0)),
            pl.BlockSpec((2, _R, 64), lambda i: (0, i, 0)),
            pl.BlockSpec((_R, 16), lambda i: (i, 0)),
            pl.BlockSpec((1, _H1), lambda i: (0, 0)),
            pl.BlockSpec((_H1, _H2), lambda i: (0, 0)),
            pl.BlockSpec((1, _H2), lambda i: (0, 0)),
            pl.BlockSpec((1, _H2), lambda i: (0, 0)),
        ],
        out_specs=[
            pl.BlockSpec((2, _R, 64), lambda i: (0, i, 0)),
            pl.BlockSpec((2, _R), lambda i: (0, i)),
        ],
        out_shape=[
            jax.ShapeDtypeStruct((2, _NPAD, 64), jnp.float32),
            jax.ShapeDtypeStruct((2, _NPAD), jnp.float32),
        ],
    )(numa, numb, den1, bias1[None], W2, att_src2[None], att_dst2[None])


def _dense3_body(num_ref, den_ref, b2_ref, h_ref):
    inv = 1.0 / (den_ref[:, :1] + 1e-16)
    h_ref[...] = jnp.concatenate(
        [num_ref[0] * inv + b2_ref[0, :64],
         num_ref[1] * inv + b2_ref[0, 64:]], axis=1)


def _dense3(num2, den2, bias2):
    return pl.pallas_call(
        _dense3_body,
        grid=(_NPAD // _R,),
        in_specs=[
            pl.BlockSpec((2, _R, 64), lambda i: (0, i, 0)),
            pl.BlockSpec((_R, 16), lambda i: (i, 0)),
            pl.BlockSpec((1, _H2), lambda i: (0, 0)),
        ],
        out_specs=pl.BlockSpec((_R, _H2), lambda i: (i, 0)),
        out_shape=jax.ShapeDtypeStruct((_NPAD, _H2), jnp.float32),
    )(num2, den2, bias2[None])


def _make_gat_edge(width):
    uw = width // 16
    nblk = _TILE_E // _BLK
    arows = _NPAD // width
    abits = width.bit_length() - 1
    mesh = plsc.VectorSubcoreMesh(core_axis_name="c", subcore_axis_name="s")

    @functools.partial(
        pl.kernel, mesh=mesh,
        compiler_params=pltpu.CompilerParams(needs_layout_passes=False,
                                             use_tc_tiling_on_sc=False),
        out_type=[jax.ShapeDtypeStruct((2 * _NPAD, width), jnp.float32),
                  jax.ShapeDtypeStruct((_NPAD, 16), jnp.float32)],
        scratch_types=[
            pltpu.VMEM((arows, width), jnp.float32),
            pltpu.VMEM((arows, width), jnp.float32),
            pltpu.VMEM((_BLK,), jnp.int32),
            pltpu.VMEM((_G, _KB), jnp.int32),
            pltpu.VMEM((_G, _KB), jnp.int32),
            pltpu.VMEM((_BLK,), jnp.float32),
            pltpu.VMEM((_BLK, 16), jnp.float32),
            pltpu.VMEM((_BLK, width), jnp.float32),
            pltpu.VMEM_SHARED((_NPAD, width), jnp.float32),
            pltpu.VMEM_SHARED((_NPAD, 16), jnp.float32),
            pltpu.SemaphoreType.DMA,
            pltpu.SemaphoreType.DMA,
        ],
    )
    def k(g_hbm, sd_hbm, out_hbm, den_hbm,
          as_v, ad_v, pi_v, si_v, di_v, w_v, wc_v, rows_v, acc_sh, den_sh,
          sem, sem2):
        c = lax.axis_index("c")
        s = lax.axis_index("s")

        def zero_body(r, _):
            for rr in range(4):
                for u in range(uw):
                    rows_v[4 * r + rr, pl.ds(u * 16, 16)] = jnp.zeros((16,), jnp.float32)
                wc_v[4 * r + rr, pl.ds(0, 16)] = jnp.zeros((16,), jnp.float32)
            return 0
        lax.fori_loop(0, _BLK // 4, zero_body, 0)
        pltpu.sync_copy(rows_v, acc_sh.at[pl.ds(s * _ROWS_T, _BLK)])
        pltpu.sync_copy(rows_v.at[pl.ds(0, _ROWS_T - _BLK)],
                        acc_sh.at[pl.ds(s * _ROWS_T + _BLK, _ROWS_T - _BLK)])
        pltpu.sync_copy(wc_v, den_sh.at[pl.ds(s * _ROWS_T, _BLK)])
        pltpu.sync_copy(wc_v.at[pl.ds(0, _ROWS_T - _BLK)],
                        den_sh.at[pl.ds(s * _ROWS_T + _BLK, _ROWS_T - _BLK)])
        pltpu.sync_copy(g_hbm.at[pl.ds(2 * _NPAD, arows)], as_v)
        pltpu.sync_copy(g_hbm.at[pl.ds(2 * _NPAD + arows, arows)], ad_v)
        plsc.subcore_barrier()

        def block(b, _):
            base = s * _TILE_E + b * _BLK
            pltpu.sync_copy(sd_hbm.at[pl.ds(base, _BLK)], pi_v)
            for j in range(_BLK // 16):
                p = pi_v[pl.ds(j * 16, 16)]
                sidx = lax.shift_right_logical(p, 14)
                didx = lax.bitwise_and(p, 16383)
                gi, l16 = j // 8, (j % 8) * 16
                di_v[gi, pl.ds(l16, 16)] = didx
                lo = lax.bitwise_and(sidx, width - 1)
                t = plsc.load_gather(as_v, [lax.shift_right_logical(sidx, abits), lo])
                lo = lax.bitwise_and(didx, width - 1)
                t = t + plsc.load_gather(ad_v, [lax.shift_right_logical(didx, abits), lo])
                t = jnp.where(t >= 0.0, t, t * 0.2)
                w = jnp.exp(t)
                eid = base + j * 16 + lax.broadcasted_iota(jnp.int32, (16,), 0)
                w = jnp.where(eid < _EAUG, w, 0.0)
                w_v[pl.ds(j * 16, 16)] = w
                si_v[gi, pl.ds(l16, 16)] = sidx + c * _NPAD
            copies = [pltpu.async_copy(g_hbm.at[si_v.at[gi]],
                                       rows_v.at[pl.ds(gi * _KB, _KB)], sem)
                      for gi in range(_G)]
            for cp in copies:
                cp.wait()

            def scale(r, _):
                for rr in range(4):
                    row = 4 * r + rr
                    wspl = plsc.load_gather(w_v, [jnp.full((16,), row, jnp.int32)])
                    wc_v[row, pl.ds(0, 16)] = wspl
                    for u in range(uw):
                        rows_v[row, pl.ds(u * 16, 16)] = rows_v[row, pl.ds(u * 16, 16)] * wspl
                return 0
            lax.fori_loop(0, _BLK // 4, scale, 0)
            adds = []
            for gi in range(_G):
                adds.append(pltpu.async_copy(
                    rows_v.at[pl.ds(gi * _KB, _KB)], acc_sh.at[di_v.at[gi]],
                    sem2, add=True))
                adds.append(pltpu.async_copy(
                    wc_v.at[pl.ds(gi * _KB, _KB)], den_sh.at[di_v.at[gi]],
                    sem2, add=True))
            for cp in adds:
                cp.wait()
            return 0
        lax.fori_loop(0, nblk, block, 0)
        plsc.subcore_barrier()

        pltpu.sync_copy(acc_sh.at[pl.ds(s * _ROWS_T, _ROWS_T)],
                        out_hbm.at[pl.ds(c * _NPAD + s * _ROWS_T, _ROWS_T)])

        @pl.when(c == 0)
        def _():
            pltpu.sync_copy(den_sh.at[pl.ds(s * _ROWS_T, _ROWS_T)],
                            den_hbm.at[pl.ds(s * _ROWS_T, _ROWS_T)])

    return k


_gat_edge_64 = _make_gat_edge(64)


def _make_edge_dot():
    nblk = _TILE_E2 // _KB
    mesh = plsc.VectorSubcoreMesh(core_axis_name="c", subcore_axis_name="s")

    @functools.partial(
        pl.kernel, mesh=mesh,
        compiler_params=pltpu.CompilerParams(needs_layout_passes=False),
        out_type=jax.ShapeDtypeStruct((_EPAD2,), jnp.float32),
        scratch_types=[
            pltpu.VMEM((_KB,), jnp.int32),
            pltpu.VMEM((_KB,), jnp.int32),
            pltpu.VMEM((_KB,), jnp.int32),
            pltpu.VMEM((_KB, _H2), jnp.float32),
            pltpu.VMEM((_KB, _H2), jnp.float32),
            pltpu.VMEM((_KB,), jnp.float32),
            pltpu.SemaphoreType.DMA,
        ],
    )
    def k(h_hbm, ee_hbm, out_hbm, pi_v, i0_v, i1_v, r0_v, r1_v, dots_v, sem):
        c = lax.axis_index("c")
        s = lax.axis_index("s")
        wid = s * 2 + c
        lane0 = lax.broadcasted_iota(jnp.int32, (16,), 0) == 0

        def block(b, _):
            base = wid * _TILE_E2 + b * _KB
            pltpu.sync_copy(ee_hbm.at[pl.ds(base, _KB)], pi_v)
            for j in range(_KB // 16):
                p = pi_v[pl.ds(j * 16, 16)]
                i0_v[pl.ds(j * 16, 16)] = lax.shift_right_logical(p, 14)
                i1_v[pl.ds(j * 16, 16)] = lax.bitwise_and(p, 16383)
            cp0 = pltpu.async_copy(h_hbm.at[i0_v], r0_v, sem)
            cp1 = pltpu.async_copy(h_hbm.at[i1_v], r1_v, sem)
            cp0.wait()
            cp1.wait()

            def dot(r, _):
                for rr in range(4):
                    row = 4 * r + rr
                    acc = r0_v[row, pl.ds(0, 16)] * r1_v[row, pl.ds(0, 16)]
                    for u in range(1, _H2 // 16):
                        acc = acc + r0_v[row, pl.ds(u * 16, 16)] * r1_v[row, pl.ds(u * 16, 16)]
                    t = jnp.sum(acc)
                    plsc.store_scatter(dots_v, [jnp.full((16,), row, jnp.int32)],
                                       jnp.full((16,), t, jnp.float32), mask=lane0)
                return 0
            lax.fori_loop(0, _KB // 4, dot, 0)
            for j in range(_KB // 16):
                v = dots_v[pl.ds(j * 16, 16)]
                dots_v[pl.ds(j * 16, 16)] = 1.0 / (1.0 + jnp.exp(-v))
            pltpu.sync_copy(dots_v, out_hbm.at[pl.ds(base, _KB)])
            return 0
        lax.fori_loop(0, nblk, block, 0)

    return k


_edge_dot = _make_edge_dot()


def kernel(x, edge_index, edge, W0, b0, W1, att_src1, att_dst1, bias1,
           W2, att_src2, att_dst2, bias2):
    loop = jnp.arange(_N, dtype=jnp.int32)
    src = jnp.concatenate([edge_index[0].astype(jnp.int32), loop])
    dst = jnp.concatenate([edge_index[1].astype(jnp.int32), loop])
    sd_p = jnp.pad(src * 16384 + dst, (0, _SD_PAD - _EAUG))
    x_p = jnp.pad(x, ((0, _NPAD - _N), (0, 0)))

    g1, a1 = _dense1(x_p, W0, b0, W1, att_src1, att_dst1)
    a1_rows = a1.reshape(2 * (_NPAD // 64), 64)
    table_a = jnp.concatenate([g1[0], g1[1], a1_rows], axis=0)
    table_b = jnp.concatenate([g1[2], g1[3], a1_rows], axis=0)
    num_a, den1 = _gat_edge_64(table_a, sd_p)
    num_b, _den_dup = _gat_edge_64(table_b, sd_p)
    g2, a2 = _dense2(num_a.reshape(2, _NPAD, 64), num_b.reshape(2, _NPAD, 64),
                     den1, bias1, W2, att_src2, att_dst2)
    table_2 = jnp.concatenate([g2[0], g2[1],
                               a2.reshape(2 * (_NPAD // 64), 64)], axis=0)
    num2, den2 = _gat_edge_64(table_2, sd_p)
    h2 = _dense3(num2.reshape(2, _NPAD, 64), den2, bias2)

    ee = edge[0].astype(jnp.int32) * 16384 + edge[1].astype(jnp.int32)
    feats = _edge_dot(h2, jnp.pad(ee, (0, _EPAD2 - _E)))
    return feats[:_E]
